# baseline (speedup 1.0000x reference)
import functools

import jax
import jax.numpy as jnp
from jax.experimental import pallas as pl

N = 50000
BM = 2000


def _embed_body(x_ref, e_ref, wn_ref, bn_ref, we_ref, be_ref,
                w_ref, w0_ref, w1_ref):
    nd = jnp.dot(x_ref[...], wn_ref[...],
                 preferred_element_type=jnp.float32) + bn_ref[...]
    ed = jnp.dot(e_ref[...], we_ref[...],
                 preferred_element_type=jnp.float32) + be_ref[...]
    w0_ref[...] = nd
    w1_ref[...] = ed
    w_ref[:, :32] = nd
    w_ref[:, 32:] = ed


def _embed(x, e, wn, bn, we, be):
    grid = N // BM
    return pl.pallas_call(
        _embed_body,
        grid=(grid,),
        in_specs=[
            pl.BlockSpec((BM, 256), lambda i: (i, 0)),
            pl.BlockSpec((BM, 16), lambda i: (i, 0)),
            pl.BlockSpec((256, 32), lambda i: (0, 0)),
            pl.BlockSpec((1, 32), lambda i: (0, 0)),
            pl.BlockSpec((16, 32), lambda i: (0, 0)),
            pl.BlockSpec((1, 32), lambda i: (0, 0)),
        ],
        out_specs=[
            pl.BlockSpec((BM, 64), lambda i: (i, 0)),
            pl.BlockSpec((BM, 32), lambda i: (i, 0)),
            pl.BlockSpec((BM, 32), lambda i: (i, 0)),
        ],
        out_shape=[
            jax.ShapeDtypeStruct((N, 64), jnp.float32),
            jax.ShapeDtypeStruct((N, 32), jnp.float32),
            jax.ShapeDtypeStruct((N, 32), jnp.float32),
        ],
    )(x, e, wn, bn.reshape(1, 32), we, be.reshape(1, 32))


def _layer1_body(agg_ref, w_ref, wl1_ref, wr1_ref, b1_ref, wr2_ref,
                 red_ref, r0_ref, r1_ref, r2_ref):
    pre = (
        jnp.dot(agg_ref[...], wl1_ref[...], preferred_element_type=jnp.float32)
        + b1_ref[...]
        + jnp.dot(w_ref[...], wr1_ref[...], preferred_element_type=jnp.float32)
    )
    nrm = jnp.sqrt(jnp.sum(pre * pre, axis=1, keepdims=True))
    red = pre / jnp.maximum(nrm, 1e-12)
    red = jnp.where(red > 0, red, jnp.exp(jnp.minimum(red, 0.0)) - 1.0)
    red_ref[...] = red
    r0_ref[...] = red[:, :32]
    r1_ref[...] = red[:, 32:]
    r2_ref[...] = jnp.dot(red, wr2_ref[...], preferred_element_type=jnp.float32)


def _layer1(agg, w, wl1, wr1, b1, wr2):
    grid = N // BM
    return pl.pallas_call(
        _layer1_body,
        grid=(grid,),
        in_specs=[
            pl.BlockSpec((BM, 64), lambda i: (i, 0)),
            pl.BlockSpec((BM, 64), lambda i: (i, 0)),
            pl.BlockSpec((64, 64), lambda i: (0, 0)),
            pl.BlockSpec((64, 64), lambda i: (0, 0)),
            pl.BlockSpec((1, 64), lambda i: (0, 0)),
            pl.BlockSpec((64, 16), lambda i: (0, 0)),
        ],
        out_specs=[
            pl.BlockSpec((BM, 64), lambda i: (i, 0)),
            pl.BlockSpec((BM, 32), lambda i: (i, 0)),
            pl.BlockSpec((BM, 32), lambda i: (i, 0)),
            pl.BlockSpec((BM, 16), lambda i: (i, 0)),
        ],
        out_shape=[
            jax.ShapeDtypeStruct((N, 64), jnp.float32),
            jax.ShapeDtypeStruct((N, 32), jnp.float32),
            jax.ShapeDtypeStruct((N, 32), jnp.float32),
            jax.ShapeDtypeStruct((N, 16), jnp.float32),
        ],
    )(agg, w, wl1, wr1, b1.reshape(1, 64), wr2)


def _final_body(agg2_ref, wl2_ref, b2_ref, r2_ref, out_ref):
    clust = (
        jnp.dot(agg2_ref[...], wl2_ref[...], preferred_element_type=jnp.float32)
        + b2_ref[...]
        + r2_ref[...]
    )
    m = jnp.max(clust, axis=1, keepdims=True)
    j = jax.lax.broadcasted_iota(jnp.int32, clust.shape, 1)
    cand = jnp.where(clust >= m, j, 16)
    jmin = jnp.min(cand, axis=1, keepdims=True)
    out_ref[...] = (j == jmin).astype(jnp.float32)


def _final(agg2, wl2, b2, r2):
    grid = N // BM
    return pl.pallas_call(
        _final_body,
        grid=(grid,),
        in_specs=[
            pl.BlockSpec((BM, 64), lambda i: (i, 0)),
            pl.BlockSpec((64, 16), lambda i: (0, 0)),
            pl.BlockSpec((1, 16), lambda i: (0, 0)),
            pl.BlockSpec((BM, 16), lambda i: (i, 0)),
        ],
        out_specs=pl.BlockSpec((BM, 16), lambda i: (i, 0)),
        out_shape=jax.ShapeDtypeStruct((N, 16), jnp.float32),
    )(agg2, wl2, b2.reshape(1, 16), r2)


def kernel(nodefeature, edge_feature, edge_index, Wn, bn, We, be, Wl1, Wr1,
           b1, Wl2, Wr2, b2):
    src = edge_index[0]
    dst = edge_index[1]
    w, w0, w1 = _embed(nodefeature, edge_feature, Wn, bn, We, be)
    agg = jax.ops.segment_sum(w[src], dst, num_segments=N)
    red, r0, r1, r2 = _layer1(agg, w, Wl1, Wr1, b1, Wr2)
    agg2 = jax.ops.segment_sum(red[src], dst, num_segments=N)
    return _final(agg2, Wl2, b2, r2)


# speedup vs baseline: 6.3925x; 6.3925x over previous
import functools

import jax
import jax.numpy as jnp
from jax import lax
from jax.experimental import pallas as pl
from jax.experimental.pallas import tpu as pltpu
from jax.experimental.pallas import tpu_sc as plsc

N = 50000
E = 800000
BM = 2000
NS = 16
EPT = E // NS
CHUNK = 400
STEPS = EPT // CHUNK
RPT = 3128
NPAD = NS * RPT


def _embed_body(x_ref, e_ref, wn_ref, bn_ref, we_ref, be_ref, w0_ref, w1_ref):
    w0_ref[...] = jnp.dot(x_ref[...], wn_ref[...],
                          preferred_element_type=jnp.float32) + bn_ref[...]
    w1_ref[...] = jnp.dot(e_ref[...], we_ref[...],
                          preferred_element_type=jnp.float32) + be_ref[...]


def _embed(x, e, wn, bn, we, be):
    return pl.pallas_call(
        _embed_body,
        grid=(N // BM,),
        in_specs=[
            pl.BlockSpec((BM, 256), lambda i: (i, 0)),
            pl.BlockSpec((BM, 16), lambda i: (i, 0)),
            pl.BlockSpec((256, 32), lambda i: (0, 0)),
            pl.BlockSpec((1, 32), lambda i: (0, 0)),
            pl.BlockSpec((16, 32), lambda i: (0, 0)),
            pl.BlockSpec((1, 32), lambda i: (0, 0)),
        ],
        out_specs=[
            pl.BlockSpec((BM, 32), lambda i: (i, 0)),
            pl.BlockSpec((BM, 32), lambda i: (i, 0)),
        ],
        out_shape=[
            jax.ShapeDtypeStruct((N, 32), jnp.float32),
            jax.ShapeDtypeStruct((N, 32), jnp.float32),
        ],
    )(x, e, wn, bn.reshape(1, 32), we, be.reshape(1, 32))


def _layer1_body(a0_ref, a1_ref, w0_ref, w1_ref, wl1_ref, wr1_ref, b1_ref,
                 wr2_ref, r0_ref, r1_ref, r2_ref):
    agg = jnp.concatenate([a0_ref[...], a1_ref[...]], axis=1)
    w = jnp.concatenate([w0_ref[...], w1_ref[...]], axis=1)
    pre = (
        jnp.dot(agg, wl1_ref[...], preferred_element_type=jnp.float32)
        + b1_ref[...]
        + jnp.dot(w, wr1_ref[...], preferred_element_type=jnp.float32)
    )
    nrm = jnp.sqrt(jnp.sum(pre * pre, axis=1, keepdims=True))
    red = pre / jnp.maximum(nrm, 1e-12)
    red = jnp.where(red > 0, red, jnp.exp(jnp.minimum(red, 0.0)) - 1.0)
    r0_ref[...] = red[:, :32]
    r1_ref[...] = red[:, 32:]
    r2_ref[...] = jnp.dot(red, wr2_ref[...], preferred_element_type=jnp.float32)


def _layer1(a0, a1, w0, w1, wl1, wr1, b1, wr2):
    return pl.pallas_call(
        _layer1_body,
        grid=(N // BM,),
        in_specs=[
            pl.BlockSpec((BM, 32), lambda i: (i, 0)),
            pl.BlockSpec((BM, 32), lambda i: (i, 0)),
            pl.BlockSpec((BM, 32), lambda i: (i, 0)),
            pl.BlockSpec((BM, 32), lambda i: (i, 0)),
            pl.BlockSpec((64, 64), lambda i: (0, 0)),
            pl.BlockSpec((64, 64), lambda i: (0, 0)),
            pl.BlockSpec((1, 64), lambda i: (0, 0)),
            pl.BlockSpec((64, 16), lambda i: (0, 0)),
        ],
        out_specs=[
            pl.BlockSpec((BM, 32), lambda i: (i, 0)),
            pl.BlockSpec((BM, 32), lambda i: (i, 0)),
            pl.BlockSpec((BM, 16), lambda i: (i, 0)),
        ],
        out_shape=[
            jax.ShapeDtypeStruct((N, 32), jnp.float32),
            jax.ShapeDtypeStruct((N, 32), jnp.float32),
            jax.ShapeDtypeStruct((N, 16), jnp.float32),
        ],
    )(a0, a1, w0, w1, wl1, wr1, b1.reshape(1, 64), wr2)


def _final_body(p0_ref, p1_ref, wl2_ref, b2_ref, r2_ref, out_ref):
    agg2 = jnp.concatenate([p0_ref[...], p1_ref[...]], axis=1)
    clust = (
        jnp.dot(agg2, wl2_ref[...], preferred_element_type=jnp.float32)
        + b2_ref[...]
        + r2_ref[...]
    )
    m = jnp.max(clust, axis=1, keepdims=True)
    j = lax.broadcasted_iota(jnp.int32, clust.shape, 1)
    cand = jnp.where(clust >= m, j, 16)
    jmin = jnp.min(cand, axis=1, keepdims=True)
    out_ref[...] = (j == jmin).astype(jnp.float32)


def _final(p0, p1, wl2, b2, r2):
    return pl.pallas_call(
        _final_body,
        grid=(N // BM,),
        in_specs=[
            pl.BlockSpec((BM, 32), lambda i: (i, 0)),
            pl.BlockSpec((BM, 32), lambda i: (i, 0)),
            pl.BlockSpec((64, 16), lambda i: (0, 0)),
            pl.BlockSpec((1, 16), lambda i: (0, 0)),
            pl.BlockSpec((BM, 16), lambda i: (i, 0)),
        ],
        out_specs=pl.BlockSpec((BM, 16), lambda i: (i, 0)),
        out_shape=jax.ShapeDtypeStruct((N, 16), jnp.float32),
    )(p0, p1, wl2, b2.reshape(1, 16), r2)


def _segsum_body(t0, t1, src_hbm, dst_hbm, o0, o1,
                 src_v, dst_v, rows_v, acc, sem):
    cid = lax.axis_index("c")
    sid = lax.axis_index("s")

    def zrow(i, _):
        rows_v[i, 0:16] = jnp.zeros((16,), jnp.float32)
        rows_v[i, 16:32] = jnp.zeros((16,), jnp.float32)
        return 0
    lax.fori_loop(0, CHUNK, zrow, 0)
    row0 = sid * RPT

    def zcopy(k, _):
        pltpu.sync_copy(rows_v, acc.at[pl.ds(row0 + k * CHUNK, CHUNK)])
        return 0
    lax.fori_loop(0, 7, zcopy, 0)
    pltpu.sync_copy(rows_v.at[pl.ds(0, RPT - 7 * CHUNK)],
                    acc.at[pl.ds(row0 + 7 * CHUNK, RPT - 7 * CHUNK)])
    plsc.subcore_barrier()

    def run(table, out):
        def step(it, _):
            base = sid * EPT + it * CHUNK
            pltpu.sync_copy(src_hbm.at[pl.ds(base, CHUNK)], src_v)
            pltpu.sync_copy(dst_hbm.at[pl.ds(base, CHUNK)], dst_v)
            pltpu.async_copy(table.at[src_v], rows_v, sem).wait()
            pltpu.sync_copy(rows_v, acc.at[dst_v], add=True)
            return 0
        lax.fori_loop(0, STEPS, step, 0)
        plsc.subcore_barrier()
        pltpu.sync_copy(acc.at[pl.ds(row0, RPT)], out.at[pl.ds(row0, RPT)])

    @pl.when(cid == 0)
    def _():
        run(t0, o0)

    @pl.when(cid == 1)
    def _():
        run(t1, o1)


def _sc_segsum(t0, t1, src, dst):
    mesh = plsc.VectorSubcoreMesh(core_axis_name="c", subcore_axis_name="s",
                                  num_cores=2, num_subcores=NS)
    f = pl.kernel(
        _segsum_body,
        out_type=[
            jax.ShapeDtypeStruct((NPAD, 32), jnp.float32),
            jax.ShapeDtypeStruct((NPAD, 32), jnp.float32),
        ],
        mesh=mesh,
        compiler_params=pltpu.CompilerParams(use_tc_tiling_on_sc=False),
        scratch_types=[
            pltpu.VMEM((CHUNK,), jnp.int32),
            pltpu.VMEM((CHUNK,), jnp.int32),
            pltpu.VMEM((CHUNK, 32), jnp.float32),
            pltpu.VMEM_SHARED((NPAD, 32), jnp.float32),
            pltpu.SemaphoreType.DMA,
        ],
    )
    return f(t0, t1, src, dst)


def kernel(nodefeature, edge_feature, edge_index, Wn, bn, We, be, Wl1, Wr1,
           b1, Wl2, Wr2, b2):
    src = edge_index[0]
    dst = edge_index[1]
    w0, w1 = _embed(nodefeature, edge_feature, Wn, bn, We, be)
    a0, a1 = _sc_segsum(w0, w1, src, dst)
    r0, r1, r2 = _layer1(a0, a1, w0, w1, Wl1, Wr1, b1, Wr2)
    p0, p1 = _sc_segsum(r0, r1, src, dst)
    return _final(p0, p1, Wl2, b2, r2)
